# baseline (speedup 1.0000x reference)
import functools

import jax
import jax.numpy as jnp
from jax import lax
from jax.experimental import pallas as pl
from jax.experimental.pallas import tpu as pltpu
from jax.experimental.pallas import tpu_sc as plsc

NC, NS, L = 2, 16, 16
NW = NC * NS
BATCH, HIST, D = 16384, 200, 64
N = BATCH * HIST
PER_W = N // NW
SUB = 128
CHUNK = 1024
NSUB = CHUNK // SUB
NCHUNK = PER_W // CHUNK
GROUPS = CHUNK // L
KSEG = D // L


def _rsqrt(v):
    i = plsc.bitcast(v, jnp.int32)
    i = jnp.int32(0x5F3759DF) - lax.shift_right_arithmetic(i, 1)
    y = plsc.bitcast(i, jnp.float32)
    for _ in range(3):
        y = y * (1.5 - 0.5 * v * y * y)
    return y


@functools.partial(
    pl.kernel,
    out_type=jax.ShapeDtypeStruct((N, D), jnp.float32),
    mesh=plsc.VectorSubcoreMesh(core_axis_name="c", subcore_axis_name="s"),
    scratch_types=[
        pltpu.VMEM((NSUB, SUB), jnp.int32),
        pltpu.VMEM((CHUNK, D), jnp.float32),
        pltpu.VMEM((D,), jnp.float32),
        pltpu.VMEM((D,), jnp.float32),
        pltpu.SemaphoreType.DMA,
    ],
    compiler_params=pltpu.CompilerParams(
        needs_layout_passes=False, use_tc_tiling_on_sc=False
    ),
)
def _embed_ln(x_hbm, table_hbm, gamma_hbm, beta_hbm, out_hbm,
              idx_v, rows_v, gamma_v, beta_v, sem):
    wid = lax.axis_index("s") * NC + lax.axis_index("c")
    pltpu.sync_copy(gamma_hbm, gamma_v)
    pltpu.sync_copy(beta_hbm, beta_v)
    gvecs = [gamma_v[pl.ds(k * L, L)] for k in range(KSEG)]
    bvecs = [beta_v[pl.ds(k * L, L)] for k in range(KSEG)]
    lane = lax.iota(jnp.int32, L)

    def chunk_body(c, carry):
        row_base = wid * PER_W + c * CHUNK
        idx_row = pl.multiple_of(row_base // SUB, 8)
        pltpu.sync_copy(x_hbm.at[pl.ds(idx_row, NSUB)], idx_v)
        copies = [
            pltpu.async_copy(table_hbm.at[idx_v.at[j]],
                             rows_v.at[pl.ds(j * SUB, SUB)], sem)
            for j in range(NSUB)
        ]
        for cp in copies:
            cp.wait()

        def group_body(g, carry2):
            rbase = g * L
            rows_idx = rbase + lane
            acc = jnp.zeros((L,), jnp.float32)
            acc2 = jnp.zeros((L,), jnp.float32)
            for d in range(D):
                col = jnp.full((L,), d, jnp.int32)
                v = plsc.load_gather(rows_v, [rows_idx, col])
                acc = acc + v
                acc2 = acc2 + v * v
            mean = acc * (1.0 / D)
            var = acc2 * (1.0 / D) - mean * mean
            rstd = _rsqrt(jnp.maximum(var, 0.0) + 1e-5)
            for r in range(L):
                m = mean[r]
                s = rstd[r]
                rref = rows_v.at[rbase + r]
                for k in range(KSEG):
                    seg = rref[pl.ds(k * L, L)]
                    rref[pl.ds(k * L, L)] = (seg - m) * s * gvecs[k] + bvecs[k]
            return carry2

        lax.fori_loop(0, GROUPS, group_body, 0)
        pltpu.sync_copy(rows_v, out_hbm.at[pl.ds(row_base, CHUNK)])
        return carry

    lax.fori_loop(0, NCHUNK, chunk_body, 0)


def kernel(x, table, gamma, beta):
    x2 = x.reshape(N // SUB, SUB).astype(jnp.int32)
    out = _embed_ln(x2, table, gamma, beta)
    return out.reshape(BATCH, HIST, D)


# speedup vs baseline: 2.2271x; 2.2271x over previous
import functools

import jax
import jax.numpy as jnp
from jax import lax
from jax.experimental import pallas as pl
from jax.experimental.pallas import tpu as pltpu
from jax.experimental.pallas import tpu_sc as plsc

NC, NS, L = 2, 16, 16
NW = NC * NS
BATCH, HIST, D = 16384, 200, 64
N = BATCH * HIST
PER_W = N // NW
SUB = 128
CHUNK = 512
NSUB = CHUNK // SUB
NCHUNK = PER_W // CHUNK
GROUPS = CHUNK // L
KSEG = D // L


@functools.partial(
    pl.kernel,
    out_type=jax.ShapeDtypeStruct((N, D), jnp.float32),
    mesh=plsc.VectorSubcoreMesh(core_axis_name="c", subcore_axis_name="s"),
    scratch_types=[
        pltpu.VMEM((CHUNK,), jnp.int32),
        pltpu.VMEM((CHUNK,), jnp.int32),
        pltpu.VMEM((CHUNK, D), jnp.float32),
        pltpu.VMEM((CHUNK, D), jnp.float32),
        pltpu.VMEM((D,), jnp.float32),
        pltpu.VMEM((D,), jnp.float32),
        pltpu.SemaphoreType.DMA,
        pltpu.SemaphoreType.DMA,
        pltpu.SemaphoreType.DMA,
        pltpu.SemaphoreType.DMA,
    ],
    compiler_params=pltpu.CompilerParams(
        needs_layout_passes=False, use_tc_tiling_on_sc=False
    ),
)
def _embed_ln(x_hbm, table_hbm, gamma_hbm, beta_hbm, out_hbm,
              idx0, idx1, rows0, rows1, gamma_v, beta_v,
              in0, in1, out0, out1):
    idx = (idx0, idx1)
    rows = (rows0, rows1)
    sem_in = (in0, in1)
    sem_out = (out0, out1)

    wid = lax.axis_index("s") * NC + lax.axis_index("c")
    base_w = wid * PER_W
    pltpu.sync_copy(gamma_hbm, gamma_v)
    pltpu.sync_copy(beta_hbm, beta_v)
    gvecs = [gamma_v[pl.ds(k * L, L)] for k in range(KSEG)]
    bvecs = [beta_v[pl.ds(k * L, L)] for k in range(KSEG)]

    def load_chunk(i, b):
        off = pl.multiple_of(base_w + i * CHUNK, 8)
        pltpu.sync_copy(x_hbm.at[pl.ds(off, CHUNK)], idx[b])
        for j in range(NSUB):
            pltpu.async_copy(
                table_hbm.at[idx[b].at[pl.ds(j * SUB, SUB)]],
                rows[b].at[pl.ds(j * SUB, SUB)],
                sem_in[b],
            )

    def wait_gather(b):
        pltpu.make_async_copy(
            table_hbm.at[pl.ds(0, CHUNK)], rows[b], sem_in[b]
        ).wait()

    def wait_out(b):
        pltpu.make_async_copy(
            rows[b], out_hbm.at[pl.ds(0, CHUNK)], sem_out[b]
        ).wait()

    def compute(b):
        rbuf = rows[b]

        def group_body(g, carry):
            r0 = g * L
            for r in range(L):
                rref = rbuf.at[r0 + r]
                v = [rref[pl.ds(k * L, L)] for k in range(KSEG)]
                s = (v[0] + v[1]) + (v[2] + v[3])
                q = [vk * vk for vk in v]
                sq = (q[0] + q[1]) + (q[2] + q[3])
                mean = jnp.sum(s) * (1.0 / D)
                ex2 = jnp.sum(sq) * (1.0 / D)
                var = jnp.maximum(ex2 - mean * mean, 0.0) + 1e-5
                bits = lax.bitcast_convert_type(var, jnp.int32)
                bits = jnp.int32(0x5F3759DF) - lax.shift_right_arithmetic(bits, 1)
                y = lax.bitcast_convert_type(bits, jnp.float32)
                for _ in range(3):
                    y = y * (1.5 - 0.5 * var * y * y)
                for k in range(KSEG):
                    rref[pl.ds(k * L, L)] = (v[k] - mean) * y * gvecs[k] + bvecs[k]
            return carry

        lax.fori_loop(0, GROUPS, group_body, 0)

    load_chunk(0, 0)

    def step(c0, carry):
        for b in range(2):
            i = c0 * 2 + b
            nb = 1 - b
            if b == 0:
                @pl.when(c0 > 0)
                def _():
                    wait_out(nb)
                load_chunk(i + 1, nb)
            else:
                @pl.when(c0 < NCHUNK // 2 - 1)
                def _():
                    wait_out(nb)
                    load_chunk(i + 1, nb)
            wait_gather(b)
            compute(b)
            off = pl.multiple_of(base_w + i * CHUNK, 8)
            pltpu.async_copy(rows[b], out_hbm.at[pl.ds(off, CHUNK)], sem_out[b])
        return carry

    lax.fori_loop(0, NCHUNK // 2, step, 0)
    wait_out(0)
    wait_out(1)


def kernel(x, table, gamma, beta):
    x1 = x.reshape(N).astype(jnp.int32)
    out = _embed_ln(x1, table, gamma, beta)
    return out.reshape(BATCH, HIST, D)
